# baseline (speedup 1.0000x reference)
import functools

import jax
import jax.numpy as jnp
from jax.experimental import pallas as pl
from jax.experimental.pallas import tpu as pltpu

B, T, D = 1, 2048, 512
K_POOL, E, TOPK = 4, 8, 2
IN_DIM, HID, OUT = D * K_POOL, 512, 1024
N = (T // K_POOL)
EPS = 1e-6


def _moe_kernel(x_ref, lnpre_ref, router_ref, shg_ref, shu_ref, shd_ref,
                eg_ref, eu_ref, ed_ref, ls_ref, lnpost_ref, out_ref,
                h16_ref, shared_ref, acc_ref, w1_ref, w2_ref, i1_ref, i2_ref):
    e = pl.program_id(0)

    @pl.when(e == 0)
    def _prologue():
        hf = x_ref[...]
        var = jnp.mean(hf * hf, axis=-1, keepdims=True)
        hf = hf * jax.lax.rsqrt(var + EPS) * lnpre_ref[...]
        h16 = hf.astype(jnp.bfloat16)
        h16_ref[...] = h16
        g = jnp.dot(h16, shg_ref[...], preferred_element_type=jnp.float32)
        u = jnp.dot(h16, shu_ref[...], preferred_element_type=jnp.float32)
        a = (jax.nn.silu(g) * u).astype(jnp.bfloat16)
        shared_ref[...] = jnp.dot(a, shd_ref[...],
                                  preferred_element_type=jnp.float32)
        logits = jnp.dot(hf, router_ref[...],
                         preferred_element_type=jnp.float32)
        m = jnp.max(logits, axis=-1, keepdims=True)
        p = jnp.exp(logits - m)
        m1 = jnp.max(p, axis=-1, keepdims=True)
        i1 = jnp.argmax(p, axis=-1, keepdims=True)
        lane = jax.lax.broadcasted_iota(jnp.int32, p.shape, 1)
        p2 = jnp.where(lane == i1, -jnp.inf, p)
        m2 = jnp.max(p2, axis=-1, keepdims=True)
        i2 = jnp.argmax(p2, axis=-1, keepdims=True)
        s = m1 + m2
        w1_ref[...] = m1 / s
        w2_ref[...] = m2 / s
        i1_ref[...] = i1.astype(jnp.int32)
        i2_ref[...] = i2.astype(jnp.int32)

    h16 = h16_ref[...]
    g = jnp.dot(h16, eg_ref[0], preferred_element_type=jnp.float32)
    u = jnp.dot(h16, eu_ref[0], preferred_element_type=jnp.float32)
    a = (jax.nn.silu(g) * u).astype(jnp.bfloat16)
    eo = jnp.dot(a, ed_ref[0], preferred_element_type=jnp.float32)
    w_e = (w1_ref[...] * (i1_ref[...] == e).astype(jnp.float32)
           + w2_ref[...] * (i2_ref[...] == e).astype(jnp.float32))
    contrib = w_e * eo

    @pl.when(e == 0)
    def _init():
        acc_ref[...] = contrib

    @pl.when(e > 0)
    def _accum():
        acc_ref[...] += contrib

    @pl.when(e == E - 1)
    def _epilogue():
        o = (shared_ref[...] + acc_ref[...]) * ls_ref[...]
        var = jnp.mean(o * o, axis=-1, keepdims=True)
        out_ref[...] = o * jax.lax.rsqrt(var + EPS) * lnpost_ref[...]


@jax.jit
def kernel(x, ln_pre_w, router_w, sh_gate, sh_up, sh_down, eg, eu, ed,
           layer_scale, ln_post_w):
    b, t, d = x.shape
    t2 = (t // K_POOL) * K_POOL
    n = t2 // K_POOL
    in_dim = d * K_POOL
    xs = x[:, :t2, :].reshape(n, in_dim)

    shg = sh_gate.astype(jnp.bfloat16)
    shu = sh_up.astype(jnp.bfloat16)
    shd = sh_down.astype(jnp.bfloat16)
    eg16 = eg.astype(jnp.bfloat16)
    eu16 = eu.astype(jnp.bfloat16)
    ed16 = ed.astype(jnp.bfloat16)

    out_dim = sh_down.shape[-1]
    hid = sh_gate.shape[-1]
    n_e = eg.shape[0]

    whole = lambda s: pl.BlockSpec(s, lambda e: (0,) * len(s))
    grid_spec = pltpu.PrefetchScalarGridSpec(
        num_scalar_prefetch=0,
        grid=(n_e,),
        in_specs=[
            whole((n, in_dim)),
            whole((1, in_dim)),
            whole((in_dim, n_e)),
            whole((in_dim, hid)),
            whole((in_dim, hid)),
            whole((hid, out_dim)),
            pl.BlockSpec((1, in_dim, hid), lambda e: (e, 0, 0)),
            pl.BlockSpec((1, in_dim, hid), lambda e: (e, 0, 0)),
            pl.BlockSpec((1, hid, out_dim), lambda e: (e, 0, 0)),
            whole((1, out_dim)),
            whole((1, out_dim)),
        ],
        out_specs=whole((n, out_dim)),
        scratch_shapes=[
            pltpu.VMEM((n, in_dim), jnp.bfloat16),
            pltpu.VMEM((n, out_dim), jnp.float32),
            pltpu.VMEM((n, out_dim), jnp.float32),
            pltpu.VMEM((n, 1), jnp.float32),
            pltpu.VMEM((n, 1), jnp.float32),
            pltpu.VMEM((n, 1), jnp.int32),
            pltpu.VMEM((n, 1), jnp.int32),
        ],
    )
    out = pl.pallas_call(
        _moe_kernel,
        grid_spec=grid_spec,
        out_shape=jax.ShapeDtypeStruct((n, out_dim), jnp.float32),
        compiler_params=pltpu.CompilerParams(
            dimension_semantics=("arbitrary",),
        ),
    )(xs, ln_pre_w.reshape(1, in_dim), router_w, shg, shu, shd,
      eg16, eu16, ed16, layer_scale.reshape(1, out_dim),
      ln_post_w.reshape(1, out_dim))
    return out.reshape(b, n, out_dim)


# speedup vs baseline: 1.7239x; 1.7239x over previous
import functools

import jax
import jax.numpy as jnp
from jax.experimental import pallas as pl
from jax.experimental.pallas import tpu as pltpu

B, T, D = 1, 2048, 512
K_POOL, E, TOPK = 4, 8, 2
IN_DIM, HID, OUT = D * K_POOL, 512, 1024
N = (T // K_POOL)
EPS = 1e-6


def _moe_kernel(x_ref, lnpre_ref, router_ref, shg_ref, shu_ref, shd_ref,
                eg_ref, eu_ref, ed_ref, ls_ref, lnpost_ref, out_ref,
                h16_ref, shared_ref, acc_ref, w1_ref, w2_ref, i1_ref, i2_ref):
    e = pl.program_id(0)

    @pl.when(e == 0)
    def _prologue():
        hf = x_ref[...]
        var = jnp.mean(hf * hf, axis=-1, keepdims=True)
        hf = hf * jax.lax.rsqrt(var + EPS) * lnpre_ref[...]
        h16 = hf.astype(jnp.bfloat16)
        h16_ref[...] = h16
        g = jnp.dot(h16, shg_ref[...].astype(jnp.bfloat16),
                    preferred_element_type=jnp.float32)
        u = jnp.dot(h16, shu_ref[...].astype(jnp.bfloat16),
                    preferred_element_type=jnp.float32)
        a = (jax.nn.silu(g) * u).astype(jnp.bfloat16)
        shared_ref[...] = jnp.dot(a, shd_ref[...].astype(jnp.bfloat16),
                                  preferred_element_type=jnp.float32)
        logits = jnp.dot(hf, router_ref[...],
                         preferred_element_type=jnp.float32)
        m = jnp.max(logits, axis=-1, keepdims=True)
        p = jnp.exp(logits - m)
        m1 = jnp.max(p, axis=-1, keepdims=True)
        i1 = jnp.argmax(p, axis=-1, keepdims=True)
        lane = jax.lax.broadcasted_iota(jnp.int32, p.shape, 1)
        p2 = jnp.where(lane == i1, -jnp.inf, p)
        m2 = jnp.max(p2, axis=-1, keepdims=True)
        i2 = jnp.argmax(p2, axis=-1, keepdims=True)
        s = m1 + m2
        w1_ref[...] = m1 / s
        w2_ref[...] = m2 / s
        i1_ref[...] = i1.astype(jnp.int32)
        i2_ref[...] = i2.astype(jnp.int32)

    h16 = h16_ref[...]
    g = jnp.dot(h16, eg_ref[0].astype(jnp.bfloat16),
                preferred_element_type=jnp.float32)
    u = jnp.dot(h16, eu_ref[0].astype(jnp.bfloat16),
                preferred_element_type=jnp.float32)
    a = (jax.nn.silu(g) * u).astype(jnp.bfloat16)
    eo = jnp.dot(a, ed_ref[0].astype(jnp.bfloat16),
                 preferred_element_type=jnp.float32)
    w_e = (w1_ref[...] * (i1_ref[...] == e).astype(jnp.float32)
           + w2_ref[...] * (i2_ref[...] == e).astype(jnp.float32))
    contrib = w_e * eo

    @pl.when(e == 0)
    def _init():
        acc_ref[...] = contrib

    @pl.when(e > 0)
    def _accum():
        acc_ref[...] += contrib

    @pl.when(e == E - 1)
    def _epilogue():
        o = (shared_ref[...] + acc_ref[...]) * ls_ref[...]
        var = jnp.mean(o * o, axis=-1, keepdims=True)
        out_ref[...] = o * jax.lax.rsqrt(var + EPS) * lnpost_ref[...]


@jax.jit
def kernel(x, ln_pre_w, router_w, sh_gate, sh_up, sh_down, eg, eu, ed,
           layer_scale, ln_post_w):
    b, t, d = x.shape
    t2 = (t // K_POOL) * K_POOL
    n = t2 // K_POOL
    in_dim = d * K_POOL
    xs = x[:, :t2, :].reshape(n, in_dim)

    out_dim = sh_down.shape[-1]
    hid = sh_gate.shape[-1]
    n_e = eg.shape[0]

    whole = lambda s: pl.BlockSpec(s, lambda e: (0,) * len(s))
    grid_spec = pltpu.PrefetchScalarGridSpec(
        num_scalar_prefetch=0,
        grid=(n_e,),
        in_specs=[
            whole((n, in_dim)),
            whole((1, in_dim)),
            whole((in_dim, n_e)),
            whole((in_dim, hid)),
            whole((in_dim, hid)),
            whole((hid, out_dim)),
            pl.BlockSpec((1, in_dim, hid), lambda e: (e, 0, 0)),
            pl.BlockSpec((1, in_dim, hid), lambda e: (e, 0, 0)),
            pl.BlockSpec((1, hid, out_dim), lambda e: (e, 0, 0)),
            whole((1, out_dim)),
            whole((1, out_dim)),
        ],
        out_specs=whole((n, out_dim)),
        scratch_shapes=[
            pltpu.VMEM((n, in_dim), jnp.bfloat16),
            pltpu.VMEM((n, out_dim), jnp.float32),
            pltpu.VMEM((n, out_dim), jnp.float32),
            pltpu.VMEM((n, 1), jnp.float32),
            pltpu.VMEM((n, 1), jnp.float32),
            pltpu.VMEM((n, 1), jnp.int32),
            pltpu.VMEM((n, 1), jnp.int32),
        ],
    )
    out = pl.pallas_call(
        _moe_kernel,
        grid_spec=grid_spec,
        out_shape=jax.ShapeDtypeStruct((n, out_dim), jnp.float32),
        compiler_params=pltpu.CompilerParams(
            dimension_semantics=("arbitrary",),
        ),
    )(xs, ln_pre_w.reshape(1, in_dim), router_w, sh_gate, sh_up, sh_down,
      eg, eu, ed, layer_scale.reshape(1, out_dim),
      ln_post_w.reshape(1, out_dim))
    return out.reshape(b, n, out_dim)
